# Initial kernel scaffold; baseline (speedup 1.0000x reference)
import jax
import jax.numpy as jnp
from jax.experimental import pallas as pl


def kernel(h0, W1, b1, W2, b2, Wm, bm, Wt, bt, Wp, bp, Em, Et, Ep, Wk, bk):
    raise NotImplementedError("write your pallas kernel here")


# speedup vs baseline: 1.2519x; 1.2519x over previous
import functools

import jax
import jax.numpy as jnp
from jax.experimental import pallas as pl
from jax.experimental.pallas import tpu as pltpu

_N, _IN_DIM, _HID, _K, _TOP_M = 16384, 512, 256, 1024, 512
_TN = 1024


def _norm_body(em_ref, et_ref, ep_ref, om_ref, ot_ref, op_ref):
    for r, o in ((em_ref, om_ref), (et_ref, ot_ref), (ep_ref, op_ref)):
        e = r[...]
        n = jnp.sqrt(jnp.sum(e * e, axis=-1, keepdims=True))
        o[...] = e / (n + 1e-8)


def _normalize_codebooks(Em, Et, Ep):
    out = jax.ShapeDtypeStruct((_K, _HID), jnp.float32)
    return pl.pallas_call(
        _norm_body,
        out_shape=(out, out, out),
    )(Em, Et, Ep)


def _main_body(h0_ref, W1_ref, b1_ref, W2_ref, b2_ref,
               Wm_ref, bm_ref, Wt_ref, bt_ref, Wp_ref, bp_ref,
               Enm_ref, Ent_ref, Enp_ref, Wk_ref, bk_ref,
               km_ref, kt_ref, kp_ref, alpha_ref):
    H = jax.nn.relu(jnp.dot(h0_ref[...], W1_ref[...]) + b1_ref[...])
    H = jax.nn.relu(jnp.dot(H, W2_ref[...]) + b2_ref[...])
    for W_ref, b_ref, En_ref, out_ref in (
            (Wm_ref, bm_ref, Enm_ref, km_ref),
            (Wt_ref, bt_ref, Ent_ref, kt_ref),
            (Wp_ref, bp_ref, Enp_ref, kp_ref)):
        z = jnp.dot(H, W_ref[...]) + b_ref[...]
        nrm = jnp.sqrt(jnp.sum(z * z, axis=-1, keepdims=True))
        zn = z / (nrm + 1e-8)
        dist = jax.lax.dot_general(
            zn, En_ref[...], (((1,), (1,)), ((), ())))
        m = jnp.min(dist, axis=1, keepdims=True)
        iota = jax.lax.broadcasted_iota(jnp.int32, dist.shape, 1)
        idx = jnp.min(jnp.where(dist == m, iota, _K), axis=1, keepdims=True)
        out_ref[...] = idx
    xk = jnp.dot(H, Wk_ref[...]) + bk_ref[...]
    alpha_ref[...] = jax.nn.sigmoid(xk)


def _fused_dense(h0, W1, b1, W2, b2, Wm, bm, Wt, bt, Wp, bp,
                 Enm, Ent, Enp, Wk, bk):
    grid = (_N // _TN,)
    row = lambda i: (i, 0)
    rep = lambda i: (0, 0)
    col_i32 = jax.ShapeDtypeStruct((_N, 1), jnp.int32)
    col_f32 = jax.ShapeDtypeStruct((_N, 1), jnp.float32)
    in_specs = [
        pl.BlockSpec((_TN, _IN_DIM), row),
        pl.BlockSpec((_IN_DIM, _HID), rep),
        pl.BlockSpec((1, _HID), rep),
        pl.BlockSpec((_HID, _HID), rep),
        pl.BlockSpec((1, _HID), rep),
    ]
    for _ in range(3):
        in_specs += [pl.BlockSpec((_HID, _HID), rep),
                     pl.BlockSpec((1, _HID), rep)]
    in_specs += [pl.BlockSpec((_K, _HID), rep)] * 3
    in_specs += [pl.BlockSpec((_HID, 1), rep),
                 pl.BlockSpec((1, 1), rep)]
    out_specs = [pl.BlockSpec((_TN, 1), row)] * 4
    km, kt, kp, alpha = pl.pallas_call(
        _main_body,
        grid=grid,
        in_specs=in_specs,
        out_specs=out_specs,
        out_shape=(col_i32, col_i32, col_i32, col_f32),
    )(h0, W1, b1.reshape(1, _HID), W2, b2.reshape(1, _HID),
      Wm, bm.reshape(1, _HID), Wt, bt.reshape(1, _HID),
      Wp, bp.reshape(1, _HID), Enm, Ent, Enp,
      Wk, bk.reshape(1, 1))
    return km, kt, kp, alpha


def kernel(h0, W1, b1, W2, b2, Wm, bm, Wt, bt, Wp, bp, Em, Et, Ep, Wk, bk):
    Enm, Ent, Enp = _normalize_codebooks(Em, Et, Ep)
    km, kt, kp, alpha = _fused_dense(
        h0, W1, b1, W2, b2, Wm, bm, Wt, bt, Wp, bp, Enm, Ent, Enp, Wk, bk)
    codes = jnp.concatenate([km, kt, kp], axis=1)
    _, key_idx = jax.lax.top_k(alpha[:, 0], _TOP_M)
    key_idx = jnp.clip(key_idx, 0, _N - 1)
    return (codes, key_idx, codes[key_idx])
